# baseline (speedup 1.0000x reference)
import functools

import jax
import jax.numpy as jnp
from jax import lax
from jax.experimental import pallas as pl
from jax.experimental.pallas import tpu as pltpu
from jax.experimental.pallas import tpu_sc as plsc

_N = 10000
_E = 320000
_D = 128
_NA = 4
_INV_SQRT_DEG = 1.0 / (32.0 ** 0.5)
_LOG2 = 0.6931471805599453

_NC = 2
_NS = 16
_NP = 10240
_NW = _NC * _NS
_EPW = _E // _NW
_C = 80
_C2 = _C // 2
_NCHUNK = _EPW // _C

_PREC = jax.lax.Precision.DEFAULT


def _dot(a, b):
    return jax.lax.dot(a, b, precision=_PREC, preferred_element_type=jnp.float32)


def _ssp(x):
    return jnp.maximum(x, 0.0) + jnp.log(1.0 + jnp.exp(-jnp.abs(x))) - _LOG2


def _rtne_bf16_bits(x):
    b = lax.bitcast_convert_type(x, jnp.int32)
    b = b + 0x7FFF + lax.bitwise_and(lax.shift_right_logical(b, 16), 1)
    return b


def _pack_bf16_pair(lo, hi):
    lo_bits = lax.shift_right_logical(_rtne_bf16_bits(lo), 16)
    hi_bits = lax.bitwise_and(_rtne_bf16_bits(hi), -65536)
    return lax.bitwise_or(lo_bits, hi_bits)


def _pack_halves(x):
    return _pack_bf16_pair(x[:, :_D // 2], x[:, _D // 2:])


_BE = 3200
_BN = 2000


def _dot_t(at, b):
    return jax.lax.dot_general(at, b, (((0,), (0,)), ((), ())),
                               precision=_PREC,
                               preferred_element_type=jnp.float32)


def _edge_wa(eet, eat, wm1t, wm2, wat):
    u = _dot(wm1t, eet)
    u = u * (1.0 / (1.0 + jnp.exp(-u)))
    return _dot_t(u, wm2) * _dot_t(eat, wat)


def _edge_body(eel_ref, eal_ref, eeh_ref, eah_ref, wm1_ref, wm2_ref, wat_ref,
               wa_ref):
    xl = _edge_wa(eel_ref[...], eal_ref[...], wm1_ref[...], wm2_ref[...],
                  wat_ref[...])
    xh = _edge_wa(eeh_ref[...], eah_ref[...], wm1_ref[...], wm2_ref[...],
                  wat_ref[...])
    wa_ref[...] = _pack_bf16_pair(xl, xh)


def _edge_weights(eet, eat, wm1, wm2, wat):
    grid = _E // 2 // _BE
    return pl.pallas_call(
        _edge_body,
        grid=(grid,),
        in_specs=[
            pl.BlockSpec((8, _BE), lambda i: (0, i)),
            pl.BlockSpec((_NA, _BE), lambda i: (0, i)),
            pl.BlockSpec((8, _BE), lambda i: (0, i + _E // 2 // _BE)),
            pl.BlockSpec((_NA, _BE), lambda i: (0, i + _E // 2 // _BE)),
            pl.BlockSpec((8, 8), lambda i: (0, 0)),
            pl.BlockSpec((8, _D), lambda i: (0, 0)),
            pl.BlockSpec((_NA, _D), lambda i: (0, 0)),
        ],
        out_specs=pl.BlockSpec((_BE, _D), lambda i: (i, 0)),
        out_shape=jax.ShapeDtypeStruct((_E // 2, _D), jnp.int32),
    )(eet, eat, eet, eat, wm1, wm2, wat)


def _self_conn(nf, at, wsc4_ref):
    acc = _dot(nf * at[:, 0:1], wsc4_ref[0])
    for a2 in range(1, _NA):
        acc = acc + _dot(nf * at[:, a2:a2 + 1], wsc4_ref[a2])
    return acc


def _node_a_body(nf_ref, at_ref, w1_ref, wsc_ref, h_ref, sc_ref):
    nf = nf_ref[...]
    h_ref[...] = _dot(nf, w1_ref[...])
    sc_ref[...] = _self_conn(nf, at_ref[...], wsc_ref)


def _node_a(nf, at, w1, wsc4):
    grid = _N // _BN
    return pl.pallas_call(
        _node_a_body,
        grid=(grid,),
        in_specs=[
            pl.BlockSpec((_BN, _D), lambda i: (i, 0)),
            pl.BlockSpec((_BN, _NA), lambda i: (i, 0)),
            pl.BlockSpec((_D, _D), lambda i: (0, 0)),
            pl.BlockSpec((_NA, _D, _D), lambda i: (0, 0, 0)),
        ],
        out_specs=[
            pl.BlockSpec((_BN, _D), lambda i: (i, 0)),
            pl.BlockSpec((_BN, _D), lambda i: (i, 0)),
        ],
        out_shape=[
            jax.ShapeDtypeStruct((_N, _D), jnp.float32),
            jax.ShapeDtypeStruct((_N, _D), jnp.float32),
        ],
    )(nf, at, w1, wsc4)


def _comb_body(sc_ref, g0_ref, g1_ref, w2_ref, at_ref, w1n_ref, wscn_ref,
               h_ref, scn_ref):
    agg = _dot(g0_ref[0] + g1_ref[0], w2_ref[...]) * _INV_SQRT_DEG
    nf = _ssp(sc_ref[...] + agg)
    h_ref[...] = _dot(nf, w1n_ref[...])
    scn_ref[...] = _self_conn(nf, at_ref[...], wscn_ref)


def _combine_next(sc, agg, w2, at, w1n, wsc4n):
    grid = _N // _BN
    return pl.pallas_call(
        _comb_body,
        grid=(grid,),
        in_specs=[
            pl.BlockSpec((_BN, _D), lambda i: (i, 0)),
            pl.BlockSpec((1, _BN, _D), lambda i: (0, i, 0)),
            pl.BlockSpec((1, _BN, _D), lambda i: (1, i, 0)),
            pl.BlockSpec((_D, _D), lambda i: (0, 0)),
            pl.BlockSpec((_BN, _NA), lambda i: (i, 0)),
            pl.BlockSpec((_D, _D), lambda i: (0, 0)),
            pl.BlockSpec((_NA, _D, _D), lambda i: (0, 0, 0)),
        ],
        out_specs=[
            pl.BlockSpec((_BN, _D), lambda i: (i, 0)),
            pl.BlockSpec((_BN, _D), lambda i: (i, 0)),
        ],
        out_shape=[
            jax.ShapeDtypeStruct((_N, _D), jnp.float32),
            jax.ShapeDtypeStruct((_N, _D), jnp.float32),
        ],
    )(sc, agg, agg, w2, at, w1n, wsc4n)


def _final_body(sc_ref, g0_ref, g1_ref, w2_ref, out_ref):
    agg = _dot(g0_ref[0] + g1_ref[0], w2_ref[...]) * _INV_SQRT_DEG
    out_ref[...] = _ssp(sc_ref[...] + agg)


def _combine_final(sc, agg, w2):
    grid = _N // _BN
    return pl.pallas_call(
        _final_body,
        grid=(grid,),
        in_specs=[
            pl.BlockSpec((_BN, _D), lambda i: (i, 0)),
            pl.BlockSpec((1, _BN, _D), lambda i: (0, i, 0)),
            pl.BlockSpec((1, _BN, _D), lambda i: (1, i, 0)),
            pl.BlockSpec((_D, _D), lambda i: (0, 0)),
        ],
        out_specs=pl.BlockSpec((_BN, _D), lambda i: (i, 0)),
        out_shape=jax.ShapeDtypeStruct((_N, _D), jnp.float32),
    )(sc, agg, agg, w2)


_MASK_HI = -65536


def _sc_body(h_hbm, wa_hbm, idx_hbm, zero_hbm, out_hbm,
             idx_v, srcb0, srcb1, dstb0, dstb1,
             rows0, rows1, wav0, wav1, agg_sh,
             g0, g1, w0, w1, s0, s1):
    c = lax.axis_index("c")
    s = lax.axis_index("s")
    wid = c * _NS + s
    rows_per_tile = _NP // _NS
    pltpu.sync_copy(zero_hbm.at[pl.ds(s * rows_per_tile, rows_per_tile)],
                    agg_sh.at[pl.ds(s * rows_per_tile, rows_per_tile)])
    pltpu.sync_copy(idx_hbm.at[wid], idx_v)
    plsc.subcore_barrier()

    pbase = wid * (_EPW // 2)
    bufs = ((srcb0, dstb0, rows0, wav0, g0, w0, s0),
            (srcb1, dstb1, rows1, wav1, g1, w1, s1))

    def issue(t, b):
        srcb, dstb, rows, wav, gs, ws, ss = bufs[b]
        for j in range(_C // 16):
            sl = pl.ds(j * 16, 16)
            x = idx_v[t, sl]
            srcb[sl] = lax.shift_right_logical(x, 16)
            dstb[sl] = lax.bitwise_and(x, 0xffff)
        pltpu.async_copy(h_hbm.at[srcb], rows, gs)
        pltpu.async_copy(wa_hbm.at[pl.ds(pbase + t * _C2, _C2)], wav, ws)

    def process(t, b):
        srcb, dstb, rows, wav, gs, ws, ss = bufs[b]
        pltpu.make_async_copy(h_hbm.at[srcb], rows, gs).wait()
        pltpu.make_async_copy(wa_hbm.at[pl.ds(0, _C2)], wav, ws).wait()

        @plsc.parallel_loop(0, _C2, unroll=2)
        def _mul(i):
            for j in range(_D // 16):
                sl = pl.ds(j * 16, 16)
                xw = wav[i, sl]
                bc = lambda v: lax.bitcast_convert_type(v, jnp.float32)
                rows[i, sl] = rows[i, sl] * bc(lax.shift_left(xw, 16))
                rows[_C2 + i, sl] = (rows[_C2 + i, sl] *
                                     bc(lax.bitwise_and(xw, _MASK_HI)))

        pltpu.sync_copy(rows, agg_sh.at[dstb], add=True)

    issue(0, 0)
    issue(1, 1)

    @pl.loop(0, (_NCHUNK - 1) // 2)
    def _pair(p):
        t0 = 2 * p
        process(t0, 0)
        issue(t0 + 2, 0)
        process(t0 + 1, 1)

        @pl.when(t0 + 3 < _NCHUNK)
        def _():
            issue(t0 + 3, 1)

    process(_NCHUNK - 1, 0)

    plsc.subcore_barrier()
    pltpu.sync_copy(agg_sh.at[pl.ds(s * rows_per_tile, rows_per_tile)],
                    out_hbm.at[c, pl.ds(s * rows_per_tile, rows_per_tile)])


@functools.cache
def _sc_message_pass_fn():
    return pl.kernel(
        _sc_body,
        out_type=jax.ShapeDtypeStruct((_NC, _NP, _D), jnp.float32),
        mesh=plsc.VectorSubcoreMesh(core_axis_name="c", subcore_axis_name="s",
                                    num_cores=_NC, num_subcores=_NS),
        scratch_types=[
            pltpu.VMEM((_NCHUNK, _C), jnp.int32),
            pltpu.VMEM((_C,), jnp.int32),
            pltpu.VMEM((_C,), jnp.int32),
            pltpu.VMEM((_C,), jnp.int32),
            pltpu.VMEM((_C,), jnp.int32),
            pltpu.VMEM((_C, _D), jnp.float32),
            pltpu.VMEM((_C, _D), jnp.float32),
            pltpu.VMEM((_C2, _D), jnp.int32),
            pltpu.VMEM((_C2, _D), jnp.int32),
            pltpu.VMEM_SHARED((_NP, _D), jnp.float32),
            pltpu.SemaphoreType.DMA,
            pltpu.SemaphoreType.DMA,
            pltpu.SemaphoreType.DMA,
            pltpu.SemaphoreType.DMA,
            pltpu.SemaphoreType.DMA,
            pltpu.SemaphoreType.DMA,
        ],
    )


def _sc_message_pass(h, wa, idx, zeros):
    return _sc_message_pass_fn()(h, wa, idx, zeros)


def kernel(node_features, node_attrs, edge_index, edge_embedding, edge_attrs,
           W1_0, Wm1_0, Wm2_0, Wattr_0, Wsc_0, W2_0,
           W1_1, Wm1_1, Wm2_1, Wattr_1, Wsc_1, W2_1):
    packed = (edge_index[0] << 16) | edge_index[1]
    idx = (packed.reshape(2, _NW, _NCHUNK, _C2)
           .transpose(1, 2, 0, 3).reshape(_NW, _NCHUNK, _C))

    wsc4_0 = Wsc_0.reshape(_D, _NA, _D).transpose(1, 0, 2)
    wsc4_1 = Wsc_1.reshape(_D, _NA, _D).transpose(1, 0, 2)

    eet = edge_embedding.T
    eat = edge_attrs.T
    wa0 = _edge_weights(eet, eat, Wm1_0.T, Wm2_0, Wattr_0)
    wa1 = _edge_weights(eet, eat, Wm1_1.T, Wm2_1, Wattr_1)

    zeros = jnp.zeros((_NP, _D), jnp.float32)

    h0, sc0 = _node_a(node_features, node_attrs, W1_0, wsc4_0)
    agg0 = _sc_message_pass(h0, wa0, idx, zeros)
    h1, sc1 = _combine_next(sc0, agg0, W2_0, node_attrs, W1_1, wsc4_1)
    agg1 = _sc_message_pass(h1, wa1, idx, zeros)
    return _combine_final(sc1, agg1, W2_1)


# speedup vs baseline: 1.0189x; 1.0189x over previous
import functools

import jax
import jax.numpy as jnp
from jax import lax
from jax.experimental import pallas as pl
from jax.experimental.pallas import tpu as pltpu
from jax.experimental.pallas import tpu_sc as plsc

_N = 10000
_E = 320000
_D = 128
_NA = 4
_INV_SQRT_DEG = 1.0 / (32.0 ** 0.5)
_LOG2 = 0.6931471805599453

_NC = 2
_NS = 16
_NP = 10240
_NW = _NC * _NS
_EPW = _E // _NW
_C = 80
_C2 = _C // 2
_NCHUNK = _EPW // _C

_PREC = jax.lax.Precision.DEFAULT


def _dot(a, b):
    return jax.lax.dot(a, b, precision=_PREC, preferred_element_type=jnp.float32)


def _ssp(x):
    return jnp.maximum(x, 0.0) + jnp.log(1.0 + jnp.exp(-jnp.abs(x))) - _LOG2


def _rtne_bf16_bits(x):
    b = lax.bitcast_convert_type(x, jnp.int32)
    b = b + 0x7FFF + lax.bitwise_and(lax.shift_right_logical(b, 16), 1)
    return b


def _pack_bf16_pair(lo, hi):
    lo_bits = lax.shift_right_logical(_rtne_bf16_bits(lo), 16)
    hi_bits = lax.bitwise_and(_rtne_bf16_bits(hi), -65536)
    return lax.bitwise_or(lo_bits, hi_bits)


def _pack_halves(x):
    return _pack_bf16_pair(x[:, :_D // 2], x[:, _D // 2:])


_BE = 3200
_BN = 2000


def _edge_wa_t(eet, eat, wm1t, wm2t, watt):
    u = _dot(wm1t, eet)
    u = u * (1.0 / (1.0 + jnp.exp(-u)))
    return _dot(wm2t, u) * _dot(watt, eat)


def _edge_body(eel_ref, eal_ref, eeh_ref, eah_ref, wm1_ref, wm2_ref, wat_ref,
               wa_ref):
    xl = _edge_wa_t(eel_ref[...], eal_ref[...], wm1_ref[...], wm2_ref[...],
                    wat_ref[...])
    xh = _edge_wa_t(eeh_ref[...], eah_ref[...], wm1_ref[...], wm2_ref[...],
                    wat_ref[...])
    wa_ref[...] = jnp.transpose(_pack_bf16_pair(xl, xh))


def _edge_weights(eet, eat, wm1, wm2, wat):
    grid = _E // 2 // _BE
    return pl.pallas_call(
        _edge_body,
        grid=(grid,),
        in_specs=[
            pl.BlockSpec((8, _BE), lambda i: (0, i)),
            pl.BlockSpec((_NA, _BE), lambda i: (0, i)),
            pl.BlockSpec((8, _BE), lambda i: (0, i + _E // 2 // _BE)),
            pl.BlockSpec((_NA, _BE), lambda i: (0, i + _E // 2 // _BE)),
            pl.BlockSpec((8, 8), lambda i: (0, 0)),
            pl.BlockSpec((_D, 8), lambda i: (0, 0)),
            pl.BlockSpec((_D, _NA), lambda i: (0, 0)),
        ],
        out_specs=pl.BlockSpec((_BE, _D), lambda i: (i, 0)),
        out_shape=jax.ShapeDtypeStruct((_E // 2, _D), jnp.int32),
    )(eet, eat, eet, eat, wm1, wm2, wat)


def _self_conn(nf, at, wsc4_ref):
    acc = _dot(nf * at[:, 0:1], wsc4_ref[0])
    for a2 in range(1, _NA):
        acc = acc + _dot(nf * at[:, a2:a2 + 1], wsc4_ref[a2])
    return acc


def _node_a_body(nf_ref, at_ref, w1_ref, wsc_ref, h_ref, sc_ref):
    nf = nf_ref[...]
    h_ref[...] = _dot(nf, w1_ref[...])
    sc_ref[...] = _self_conn(nf, at_ref[...], wsc_ref)


def _node_a(nf, at, w1, wsc4):
    grid = _N // _BN
    return pl.pallas_call(
        _node_a_body,
        grid=(grid,),
        in_specs=[
            pl.BlockSpec((_BN, _D), lambda i: (i, 0)),
            pl.BlockSpec((_BN, _NA), lambda i: (i, 0)),
            pl.BlockSpec((_D, _D), lambda i: (0, 0)),
            pl.BlockSpec((_NA, _D, _D), lambda i: (0, 0, 0)),
        ],
        out_specs=[
            pl.BlockSpec((_BN, _D), lambda i: (i, 0)),
            pl.BlockSpec((_BN, _D), lambda i: (i, 0)),
        ],
        out_shape=[
            jax.ShapeDtypeStruct((_N, _D), jnp.float32),
            jax.ShapeDtypeStruct((_N, _D), jnp.float32),
        ],
    )(nf, at, w1, wsc4)


def _comb_body(sc_ref, g0_ref, g1_ref, w2_ref, at_ref, w1n_ref, wscn_ref,
               h_ref, scn_ref):
    agg = _dot(g0_ref[0] + g1_ref[0], w2_ref[...]) * _INV_SQRT_DEG
    nf = _ssp(sc_ref[...] + agg)
    h_ref[...] = _dot(nf, w1n_ref[...])
    scn_ref[...] = _self_conn(nf, at_ref[...], wscn_ref)


def _combine_next(sc, agg, w2, at, w1n, wsc4n):
    grid = _N // _BN
    return pl.pallas_call(
        _comb_body,
        grid=(grid,),
        in_specs=[
            pl.BlockSpec((_BN, _D), lambda i: (i, 0)),
            pl.BlockSpec((1, _BN, _D), lambda i: (0, i, 0)),
            pl.BlockSpec((1, _BN, _D), lambda i: (1, i, 0)),
            pl.BlockSpec((_D, _D), lambda i: (0, 0)),
            pl.BlockSpec((_BN, _NA), lambda i: (i, 0)),
            pl.BlockSpec((_D, _D), lambda i: (0, 0)),
            pl.BlockSpec((_NA, _D, _D), lambda i: (0, 0, 0)),
        ],
        out_specs=[
            pl.BlockSpec((_BN, _D), lambda i: (i, 0)),
            pl.BlockSpec((_BN, _D), lambda i: (i, 0)),
        ],
        out_shape=[
            jax.ShapeDtypeStruct((_N, _D), jnp.float32),
            jax.ShapeDtypeStruct((_N, _D), jnp.float32),
        ],
    )(sc, agg, agg, w2, at, w1n, wsc4n)


def _final_body(sc_ref, g0_ref, g1_ref, w2_ref, out_ref):
    agg = _dot(g0_ref[0] + g1_ref[0], w2_ref[...]) * _INV_SQRT_DEG
    out_ref[...] = _ssp(sc_ref[...] + agg)


def _combine_final(sc, agg, w2):
    grid = _N // _BN
    return pl.pallas_call(
        _final_body,
        grid=(grid,),
        in_specs=[
            pl.BlockSpec((_BN, _D), lambda i: (i, 0)),
            pl.BlockSpec((1, _BN, _D), lambda i: (0, i, 0)),
            pl.BlockSpec((1, _BN, _D), lambda i: (1, i, 0)),
            pl.BlockSpec((_D, _D), lambda i: (0, 0)),
        ],
        out_specs=pl.BlockSpec((_BN, _D), lambda i: (i, 0)),
        out_shape=jax.ShapeDtypeStruct((_N, _D), jnp.float32),
    )(sc, agg, agg, w2)


_MASK_HI = -65536


def _sc_body(h_hbm, wa_hbm, idx_hbm, zero_hbm, out_hbm,
             idx_v, srcb0, srcb1, dstb0, dstb1,
             rows0, rows1, wav0, wav1, agg_sh,
             g0, g1, w0, w1, s0, s1):
    c = lax.axis_index("c")
    s = lax.axis_index("s")
    wid = c * _NS + s
    rows_per_tile = _NP // _NS
    pltpu.sync_copy(zero_hbm.at[pl.ds(s * rows_per_tile, rows_per_tile)],
                    agg_sh.at[pl.ds(s * rows_per_tile, rows_per_tile)])
    pltpu.sync_copy(idx_hbm.at[wid], idx_v)
    plsc.subcore_barrier()

    pbase = wid * (_EPW // 2)
    bufs = ((srcb0, dstb0, rows0, wav0, g0, w0, s0),
            (srcb1, dstb1, rows1, wav1, g1, w1, s1))

    def issue(t, b):
        srcb, dstb, rows, wav, gs, ws, ss = bufs[b]
        for j in range(_C // 16):
            sl = pl.ds(j * 16, 16)
            x = idx_v[t, sl]
            srcb[sl] = lax.shift_right_logical(x, 16)
            dstb[sl] = lax.bitwise_and(x, 0xffff)
        pltpu.async_copy(h_hbm.at[srcb], rows, gs)
        pltpu.async_copy(wa_hbm.at[pl.ds(pbase + t * _C2, _C2)], wav, ws)

    def process(t, b):
        srcb, dstb, rows, wav, gs, ws, ss = bufs[b]
        pltpu.make_async_copy(h_hbm.at[srcb], rows, gs).wait()
        pltpu.make_async_copy(wa_hbm.at[pl.ds(0, _C2)], wav, ws).wait()

        @plsc.parallel_loop(0, _C2, unroll=2)
        def _mul(i):
            for j in range(_D // 16):
                sl = pl.ds(j * 16, 16)
                xw = wav[i, sl]
                bc = lambda v: lax.bitcast_convert_type(v, jnp.float32)
                rows[i, sl] = rows[i, sl] * bc(lax.shift_left(xw, 16))
                rows[_C2 + i, sl] = (rows[_C2 + i, sl] *
                                     bc(lax.bitwise_and(xw, _MASK_HI)))

        pltpu.sync_copy(rows, agg_sh.at[dstb], add=True)

    issue(0, 0)
    issue(1, 1)

    @pl.loop(0, (_NCHUNK - 1) // 2)
    def _pair(p):
        t0 = 2 * p
        process(t0, 0)
        issue(t0 + 2, 0)
        process(t0 + 1, 1)

        @pl.when(t0 + 3 < _NCHUNK)
        def _():
            issue(t0 + 3, 1)

    process(_NCHUNK - 1, 0)

    plsc.subcore_barrier()
    pltpu.sync_copy(agg_sh.at[pl.ds(s * rows_per_tile, rows_per_tile)],
                    out_hbm.at[c, pl.ds(s * rows_per_tile, rows_per_tile)])


@functools.cache
def _sc_message_pass_fn():
    return pl.kernel(
        _sc_body,
        out_type=jax.ShapeDtypeStruct((_NC, _NP, _D), jnp.float32),
        mesh=plsc.VectorSubcoreMesh(core_axis_name="c", subcore_axis_name="s",
                                    num_cores=_NC, num_subcores=_NS),
        scratch_types=[
            pltpu.VMEM((_NCHUNK, _C), jnp.int32),
            pltpu.VMEM((_C,), jnp.int32),
            pltpu.VMEM((_C,), jnp.int32),
            pltpu.VMEM((_C,), jnp.int32),
            pltpu.VMEM((_C,), jnp.int32),
            pltpu.VMEM((_C, _D), jnp.float32),
            pltpu.VMEM((_C, _D), jnp.float32),
            pltpu.VMEM((_C2, _D), jnp.int32),
            pltpu.VMEM((_C2, _D), jnp.int32),
            pltpu.VMEM_SHARED((_NP, _D), jnp.float32),
            pltpu.SemaphoreType.DMA,
            pltpu.SemaphoreType.DMA,
            pltpu.SemaphoreType.DMA,
            pltpu.SemaphoreType.DMA,
            pltpu.SemaphoreType.DMA,
            pltpu.SemaphoreType.DMA,
        ],
    )


def _sc_message_pass(h, wa, idx, zeros):
    return _sc_message_pass_fn()(h, wa, idx, zeros)


def kernel(node_features, node_attrs, edge_index, edge_embedding, edge_attrs,
           W1_0, Wm1_0, Wm2_0, Wattr_0, Wsc_0, W2_0,
           W1_1, Wm1_1, Wm2_1, Wattr_1, Wsc_1, W2_1):
    packed = (edge_index[0] << 16) | edge_index[1]
    idx = (packed.reshape(2, _NW, _NCHUNK, _C2)
           .transpose(1, 2, 0, 3).reshape(_NW, _NCHUNK, _C))

    wsc4_0 = Wsc_0.reshape(_D, _NA, _D).transpose(1, 0, 2)
    wsc4_1 = Wsc_1.reshape(_D, _NA, _D).transpose(1, 0, 2)

    eet = edge_embedding.T
    eat = edge_attrs.T
    wa0 = _edge_weights(eet, eat, Wm1_0.T, Wm2_0.T, Wattr_0.T)
    wa1 = _edge_weights(eet, eat, Wm1_1.T, Wm2_1.T, Wattr_1.T)

    zeros = jnp.zeros((_NP, _D), jnp.float32)

    h0, sc0 = _node_a(node_features, node_attrs, W1_0, wsc4_0)
    agg0 = _sc_message_pass(h0, wa0, idx, zeros)
    h1, sc1 = _combine_next(sc0, agg0, W2_0, node_attrs, W1_1, wsc4_1)
    agg1 = _sc_message_pass(h1, wa1, idx, zeros)
    return _combine_final(sc1, agg1, W2_1)
